# baseline (speedup 1.0000x reference)
import functools

import jax
import jax.numpy as jnp
from jax import lax
from jax.experimental import pallas as pl
from jax.experimental.pallas import tpu as pltpu
from jax.experimental.pallas import tpu_sc as plsc

D_MODEL = 64
SCALE = 8.0

NUM_CORES = 2
NUM_SUBCORES = 16
NW = NUM_CORES * NUM_SUBCORES

B_TOTAL = 16384 * 50
N_W = B_TOTAL // NW
CHUNK = 256
SUB = 128
NSUB = CHUNK // SUB
G = N_W // CHUNK
NBUF = 2
ROW_UNROLL = 8

_mesh = plsc.VectorSubcoreMesh(core_axis_name="c", subcore_axis_name="s")


@functools.partial(
    pl.kernel,
    mesh=_mesh,
    out_type=jax.ShapeDtypeStruct((B_TOTAL, D_MODEL), jnp.float32),
    compiler_params=pltpu.CompilerParams(use_tc_tiling_on_sc=False),
    scratch_types=[
        pltpu.VMEM((N_W,), jnp.int32),
        pltpu.VMEM((NBUF, CHUNK, D_MODEL), jnp.float32),
        pltpu.VMEM((NBUF, CHUNK, D_MODEL), jnp.float32),
        pltpu.SemaphoreType.DMA,
        pltpu.SemaphoreType.DMA,
        pltpu.SemaphoreType.DMA,
        pltpu.SemaphoreType.DMA,
    ],
)
def _emb_lookup(x_hbm, table_hbm, out_hbm, idx_v, gbuf, sbuf,
                gsem0, gsem1, ssem0, ssem1):
    gsems = (gsem0, gsem1)
    ssems = (ssem0, ssem1)
    wid = lax.axis_index("s") * NUM_CORES + lax.axis_index("c")
    base = wid * N_W

    pltpu.sync_copy(x_hbm.at[pl.ds(base, N_W)], idx_v)

    def fire_gather(chunk, b):
        for s in range(NSUB):
            off = chunk * CHUNK + s * SUB
            pltpu.async_copy(
                table_hbm.at[idx_v.at[pl.ds(off, SUB)]],
                gbuf.at[b, pl.ds(s * SUB, SUB)],
                gsems[b],
            )

    def wait_gather(b):
        for s in range(NSUB):
            pltpu.make_async_copy(
                table_hbm.at[idx_v.at[pl.ds(s * SUB, SUB)]],
                gbuf.at[b, pl.ds(s * SUB, SUB)],
                gsems[b],
            ).wait()

    def wait_scatter(chunk, b):
        pltpu.make_async_copy(
            sbuf.at[b],
            out_hbm.at[pl.ds(base + chunk * CHUNK, CHUNK)],
            ssems[b],
        ).wait()

    for b in range(NBUF):
        fire_gather(jnp.int32(b), b)

    def outer(t, carry):
        for b in range(NBUF):
            cur = t * NBUF + b
            wait_gather(b)

            @pl.when(t >= 1)
            def _():
                wait_scatter(cur - NBUF, b)

            def scale_rows(r, c):
                for u in range(ROW_UNROLL):
                    for j in range(D_MODEL // 16):
                        sl = pl.ds(j * 16, 16)
                        sbuf[b, r * ROW_UNROLL + u, sl] = (
                            gbuf[b, r * ROW_UNROLL + u, sl] * SCALE)
                return c
            lax.fori_loop(0, CHUNK // ROW_UNROLL, scale_rows, 0)

            @pl.when(cur + NBUF < G)
            def _():
                fire_gather(cur + NBUF, b)

            pltpu.async_copy(
                sbuf.at[b],
                out_hbm.at[pl.ds(base + cur * CHUNK, CHUNK)],
                ssems[b],
            )
        return carry

    lax.fori_loop(0, G // NBUF, outer, 0)

    for b in range(NBUF):
        wait_scatter(G - NBUF + b, b)


def kernel(x, table):
    xf = x.reshape(-1).astype(jnp.int32)
    out = _emb_lookup(xf, table)
    return out.reshape(x.shape[0], x.shape[1], D_MODEL)
